# baseline (speedup 1.0000x reference)
import functools
import math

import jax
import jax.numpy as jnp
from jax.experimental import pallas as pl
from jax.experimental.pallas import tpu as pltpu

N = 4096
H = 256
W = 256

NSTRIP = 4
SW = W // NSTRIP
ROWS = 16
PB = ROWS * SW
NB = 128
NBANDS = H // ROWS
SQ2T = 5.2915
CLASS_SMAX = (2.0, 4.0, 6.0, 8.0)
NPAD = N + NB
CX = W * 0.5 + 0.5
CY = H * 0.5 + 0.5


def _params_kernel(p_ref, k_ref, fw_ref):
    x = p_ref[0:1, :]
    y = p_ref[1:2, :]
    sx = jnp.abs(p_ref[2:3, :])
    sy = jnp.abs(p_ref[3:4, :])
    rot = p_ref[4:5, :]
    f0 = p_ref[5:6, :]
    f1 = p_ref[6:7, :]
    f2 = p_ref[7:8, :]
    w = p_ref[8:9, :]

    mx = 0.5 * (x + 1.0) * W
    my = 0.5 * (y + 1.0) * H
    theta = jax.nn.sigmoid(rot) * (2.0 * math.pi)
    c = jnp.cos(theta)
    sn = jnp.sin(theta)
    sx2 = sx * sx
    sy2 = sy * sy
    Sxx = c * c * sx2 + sn * sn * sy2
    Sxy = c * sn * (sx2 - sy2)
    Syy = sn * sn * sx2 + c * c * sy2
    det = Sxx * Syy - Sxy * Sxy
    inv = 1.0 / (det + 1e-12)
    a = 0.5 * Syy * inv
    cc = -Sxy * inv
    b = 0.5 * Sxx * inv

    dmx = mx - CX
    dmy = my - CY
    k3 = -(2.0 * a * dmx + cc * dmy)
    k4 = -(2.0 * b * dmy + cc * dmx)
    k5 = a * dmx * dmx + b * dmy * dmy + cc * dmx * dmy

    zero = jnp.zeros_like(x)
    rows = [a, b, cc, k3, k4, k5, zero, zero]
    for i, r in enumerate(rows):
        k1 = r.astype(jnp.bfloat16)
        r1 = r - k1.astype(jnp.float32)
        k2 = r1.astype(jnp.bfloat16)
        r2 = r1 - k2.astype(jnp.float32)
        k3b = r2.astype(jnp.bfloat16)
        k_ref[i:i + 1, :] = k1
        k_ref[8 + i:9 + i, :] = k2
        k_ref[16 + i:17 + i, :] = k3b
        k_ref[24 + i:25 + i, :] = k1
        k_ref[32 + i:33 + i, :] = k2

    fw_ref[0:1, :] = f0 * w
    fw_ref[1:2, :] = f1 * w
    fw_ref[2:3, :] = f2 * w
    fw_ref[3:8, :] = jnp.concatenate([zero] * 5, axis=0)


def _raster_kernel(s_ref, k_ref, fw_ref, out_ref):
    i = pl.program_id(0)
    s = pl.program_id(1)

    pix = jax.lax.broadcasted_iota(jnp.int32, (PB, 40), 0)
    lane = jax.lax.broadcasted_iota(jnp.int32, (PB, 40), 1)
    col = (pix & (SW - 1)) + s * SW
    row = pix >> 6
    pxi = col - (W // 2)
    pyi = row + i * ROWS - (H // 2)
    qxx = pxi * pxi
    qyy = pyi * pyi
    qxy = pxi * pyi
    hxx = qxx & ~63
    hyy = qyy & ~63
    hxy = (qxy >> 6) << 6
    lxx = qxx - hxx
    lyy = qyy - hyy
    lxy = qxy - hxy
    m = lane & 7
    is_lo = lane >= 24
    fhi = jnp.where(m == 0, hxx,
          jnp.where(m == 1, hyy,
          jnp.where(m == 2, hxy,
          jnp.where(m == 3, pxi,
          jnp.where(m == 4, pyi,
          jnp.where(m == 5, 1, 0))))))
    flo = jnp.where(m == 0, lxx,
          jnp.where(m == 1, lyy,
          jnp.where(m == 2, lxy, 0)))
    Pf = jnp.where(is_lo, flo, fhi).astype(jnp.float32).astype(jnp.bfloat16)

    glc = jax.lax.broadcasted_iota(jnp.int32, (NB, 8), 0)

    acc = jnp.zeros((PB, 8), jnp.float32)
    for c in range(4):
        lo = s_ref[i, s * 8 + 2 * c]
        hi = s_ref[i, s * 8 + 2 * c + 1]
        nblk = (hi - (lo // NB) * NB + NB - 1) // NB

        def body(j, acc, lo=lo, hi=hi):
            base = pl.multiple_of((lo // NB + j) * NB, NB)
            K = k_ref[:, pl.ds(base, NB)]
            sigma = jnp.dot(Pf, K, preferred_element_type=jnp.float32)
            vals = jnp.exp(-sigma).astype(jnp.bfloat16)
            gidx = glc + base
            fwb = jnp.where((gidx >= lo) & (gidx < hi),
                            fw_ref[pl.ds(base, NB), :], jnp.bfloat16(0))
            return acc + jnp.dot(vals, fwb, preferred_element_type=jnp.float32)

        acc = jax.lax.fori_loop(0, nblk, body, acc)

    out_ref[...] = jnp.clip(acc, 0.0, 1.0).reshape(ROWS, SW, 8)


@jax.jit
def kernel(xyz, scaling, rotation, features, opacity):
    mxf = 0.5 * (xyz[:, 0] + 1.0) * W
    myf = 0.5 * (xyz[:, 1] + 1.0) * H
    s_max = jnp.maximum(jnp.abs(scaling[:, 0]), jnp.abs(scaling[:, 1]))
    cls = ((s_max > CLASS_SMAX[0]).astype(jnp.int32)
           + (s_max > CLASS_SMAX[1]).astype(jnp.int32)
           + (s_max > CLASS_SMAX[2]).astype(jnp.int32))
    Rc = jnp.array([SQ2T * sm for sm in CLASS_SMAX], jnp.float32)
    rg = Rc[cls]
    key = cls.astype(jnp.float32) * 1024.0 + myf

    y0 = jnp.arange(NBANDS, dtype=jnp.float32) * ROWS + 0.5
    y1 = y0 + (ROWS - 1)
    ckey = jnp.arange(4, dtype=jnp.float32) * 1024.0
    lo_q = ckey[None, :] + jnp.maximum(y0[:, None] - Rc[None, :], 0.0) - 1e-3
    hi_q = ckey[None, :] + jnp.minimum(y1[:, None] + Rc[None, :], 256.0) + 1e-3

    params = jnp.concatenate(
        [xyz.T, scaling.T, rotation.T, features.T, opacity.T,
         jnp.zeros((7, N), jnp.float32)], axis=0)

    strip_params = []
    scal_cols = []
    for s in range(NSTRIP):
        xs0 = s * SW + 0.5
        xs1 = xs0 + (SW - 1)
        in_s = (mxf >= xs0 - rg) & (mxf <= xs1 + rg)
        key_s = jnp.where(in_s, key, jnp.float32(3.0e7))
        order_s = jnp.argsort(key_s)
        ks = key_s[order_s]
        lo = jnp.searchsorted(ks, lo_q.ravel()).astype(jnp.int32)
        hi = jnp.searchsorted(ks, hi_q.ravel()).astype(jnp.int32)
        lo = lo.reshape(NBANDS, 4) + s * NPAD
        hi = hi.reshape(NBANDS, 4) + s * NPAD
        scal_cols += [lo[:, 0], hi[:, 0], lo[:, 1], hi[:, 1],
                      lo[:, 2], hi[:, 2], lo[:, 3], hi[:, 3]]
        ps = params[:, order_s]
        strip_params.append(jnp.concatenate(
            [ps, jnp.zeros((16, NPAD - N), jnp.float32)], axis=1))
    scal = jnp.stack(scal_cols, axis=1)
    params_all = jnp.concatenate(strip_params, axis=1)

    kcoef, fwT = pl.pallas_call(
        _params_kernel,
        out_shape=[jax.ShapeDtypeStruct((40, NSTRIP * NPAD), jnp.bfloat16),
                   jax.ShapeDtypeStruct((8, NSTRIP * NPAD), jnp.float32)],
    )(params_all)
    fw = fwT.T.astype(jnp.bfloat16)

    out = pl.pallas_call(
        _raster_kernel,
        grid_spec=pltpu.PrefetchScalarGridSpec(
            num_scalar_prefetch=1,
            grid=(NBANDS, NSTRIP),
            in_specs=[
                pl.BlockSpec((40, NSTRIP * NPAD), lambda i, s, sc: (0, 0)),
                pl.BlockSpec((NSTRIP * NPAD, 8), lambda i, s, sc: (0, 0)),
            ],
            out_specs=pl.BlockSpec((ROWS, SW, 8), lambda i, s, sc: (i, s, 0)),
        ),
        out_shape=jax.ShapeDtypeStruct((H, W, 8), jnp.float32),
    )(scal, kcoef, fw)

    img = out[:, :, :3].reshape(1, H, W, 3).transpose(0, 3, 1, 2)
    return img


# speedup vs baseline: 1.4084x; 1.4084x over previous
import functools
import math

import jax
import jax.numpy as jnp
from jax.experimental import pallas as pl
from jax.experimental.pallas import tpu as pltpu

N = 4096
H = 256
W = 256

ROWS_PER_BAND = 8
PB = ROWS_PER_BAND * W
NB = 256
NBANDS = H // ROWS_PER_BAND
SQ2T = 5.2915
CLASS_SMAX = (2.0, 4.0, 6.0, 8.0)
NPAD = N + NB
CX = W * 0.5 + 0.5
CY = H * 0.5 + 0.5


def _params_kernel(p_ref, k_ref, fw_ref):
    x = p_ref[0:1, :]
    y = p_ref[1:2, :]
    sx = jnp.abs(p_ref[2:3, :])
    sy = jnp.abs(p_ref[3:4, :])
    rot = p_ref[4:5, :]
    f0 = p_ref[5:6, :]
    f1 = p_ref[6:7, :]
    f2 = p_ref[7:8, :]
    w = p_ref[8:9, :]

    mx = 0.5 * (x + 1.0) * W
    my = 0.5 * (y + 1.0) * H
    theta = jax.nn.sigmoid(rot) * (2.0 * math.pi)
    c = jnp.cos(theta)
    sn = jnp.sin(theta)
    sx2 = sx * sx
    sy2 = sy * sy
    Sxx = c * c * sx2 + sn * sn * sy2
    Sxy = c * sn * (sx2 - sy2)
    Syy = sn * sn * sx2 + c * c * sy2
    det = Sxx * Syy - Sxy * Sxy
    inv = 1.0 / (det + 1e-12)
    a = 0.5 * Syy * inv
    cc = -Sxy * inv
    b = 0.5 * Sxx * inv

    dmx = mx - CX
    dmy = my - CY
    k3 = -(2.0 * a * dmx + cc * dmy)
    k4 = -(2.0 * b * dmy + cc * dmx)
    k5 = a * dmx * dmx + b * dmy * dmy + cc * dmx * dmy

    zero = jnp.zeros_like(x)
    rows = [a, b, cc, k3, k4, k5, zero, zero]
    for i, r in enumerate(rows):
        k1 = r.astype(jnp.bfloat16)
        r1 = r - k1.astype(jnp.float32)
        k2 = r1.astype(jnp.bfloat16)
        r2 = r1 - k2.astype(jnp.float32)
        k3b = r2.astype(jnp.bfloat16)
        k_ref[i:i + 1, :] = k1
        k_ref[8 + i:9 + i, :] = k2
        k_ref[16 + i:17 + i, :] = k3b
        k_ref[24 + i:25 + i, :] = k1
        k_ref[32 + i:33 + i, :] = k2

    fw_ref[0:1, :] = f0 * w
    fw_ref[1:2, :] = f1 * w
    fw_ref[2:3, :] = f2 * w
    fw_ref[3:8, :] = jnp.concatenate([zero] * 5, axis=0)


def _raster_kernel(s_ref, k_ref, fw_ref, out_ref):
    i = pl.program_id(0)

    pix = jax.lax.broadcasted_iota(jnp.int32, (PB, 40), 0)
    lane = jax.lax.broadcasted_iota(jnp.int32, (PB, 40), 1)
    col = pix & (W - 1)
    row = pix >> 8
    pxi = col - (W // 2)
    pyi = row + i * ROWS_PER_BAND - (H // 2)
    qxx = pxi * pxi
    qyy = pyi * pyi
    qxy = pxi * pyi
    hxx = qxx & ~63
    hyy = qyy & ~63
    hxy = (qxy >> 6) << 6
    lxx = qxx - hxx
    lyy = qyy - hyy
    lxy = qxy - hxy
    m = lane & 7
    is_lo = lane >= 24
    fhi = jnp.where(m == 0, hxx,
          jnp.where(m == 1, hyy,
          jnp.where(m == 2, hxy,
          jnp.where(m == 3, pxi,
          jnp.where(m == 4, pyi,
          jnp.where(m == 5, 1, 0))))))
    flo = jnp.where(m == 0, lxx,
          jnp.where(m == 1, lyy,
          jnp.where(m == 2, lxy, 0)))
    Pf = jnp.where(is_lo, flo, fhi).astype(jnp.float32).astype(jnp.bfloat16)

    glc = jax.lax.broadcasted_iota(jnp.int32, (NB, 8), 0)

    acc = jnp.zeros((PB, 8), jnp.float32)
    for c in range(4):
        lo = s_ref[i, 2 * c]
        hi = s_ref[i, 2 * c + 1]
        nblk = (hi - (lo // NB) * NB + NB - 1) // NB

        def body(j, acc, lo=lo, hi=hi):
            base = pl.multiple_of((lo // NB + j) * NB, NB)
            K = k_ref[:, pl.ds(base, NB)]
            sigma = jnp.dot(Pf, K, preferred_element_type=jnp.float32)
            vals = jnp.exp(-sigma).astype(jnp.bfloat16)
            gidx = glc + base
            fwb = jnp.where((gidx >= lo) & (gidx < hi),
                            fw_ref[pl.ds(base, NB), :], jnp.bfloat16(0))
            return acc + jnp.dot(vals, fwb, preferred_element_type=jnp.float32)

        acc = jax.lax.fori_loop(0, nblk, body, acc)

    out_ref[...] = jnp.clip(acc, 0.0, 1.0)


@jax.jit
def kernel(xyz, scaling, rotation, features, opacity):
    myf = 0.5 * (xyz[:, 1] + 1.0) * H
    s_max = jnp.maximum(jnp.abs(scaling[:, 0]), jnp.abs(scaling[:, 1]))
    cls = ((s_max > CLASS_SMAX[0]).astype(jnp.int32)
           + (s_max > CLASS_SMAX[1]).astype(jnp.int32)
           + (s_max > CLASS_SMAX[2]).astype(jnp.int32))
    key = cls.astype(jnp.float32) * 1024.0 + myf
    order = jnp.argsort(key)
    key_s = key[order]

    y0 = jnp.arange(NBANDS, dtype=jnp.float32) * ROWS_PER_BAND + 0.5
    y1 = y0 + (ROWS_PER_BAND - 1)
    Rc = jnp.array([SQ2T * s for s in CLASS_SMAX], jnp.float32)
    ckey = jnp.arange(4, dtype=jnp.float32) * 1024.0
    lo_q = ckey[None, :] + jnp.maximum(y0[:, None] - Rc[None, :], 0.0) - 1e-3
    hi_q = ckey[None, :] + jnp.minimum(y1[:, None] + Rc[None, :], 256.0) + 1e-3
    lo = jnp.searchsorted(key_s, lo_q.ravel()).astype(jnp.int32)
    hi = jnp.searchsorted(key_s, hi_q.ravel()).astype(jnp.int32)
    lo = lo.reshape(NBANDS, 4)
    hi = hi.reshape(NBANDS, 4)
    scal = jnp.stack([lo[:, 0], hi[:, 0], lo[:, 1], hi[:, 1],
                      lo[:, 2], hi[:, 2], lo[:, 3], hi[:, 3]], axis=1)

    params = jnp.concatenate(
        [xyz.T, scaling.T, rotation.T, features.T, opacity.T,
         jnp.zeros((7, N), jnp.float32)], axis=0)
    params = params[:, order]
    params = jnp.concatenate(
        [params, jnp.zeros((16, NPAD - N), jnp.float32)], axis=1)

    kcoef, fwT = pl.pallas_call(
        _params_kernel,
        out_shape=[jax.ShapeDtypeStruct((40, NPAD), jnp.bfloat16),
                   jax.ShapeDtypeStruct((8, NPAD), jnp.float32)],
    )(params)
    fw = fwT.T.astype(jnp.bfloat16)

    out = pl.pallas_call(
        _raster_kernel,
        grid_spec=pltpu.PrefetchScalarGridSpec(
            num_scalar_prefetch=1,
            grid=(NBANDS,),
            in_specs=[
                pl.BlockSpec((40, NPAD), lambda i, s: (0, 0)),
                pl.BlockSpec((NPAD, 8), lambda i, s: (0, 0)),
            ],
            out_specs=pl.BlockSpec((PB, 8), lambda i, s: (i, 0)),
        ),
        out_shape=jax.ShapeDtypeStruct((H * W, 8), jnp.float32),
    )(scal, kcoef, fw)

    img = out[:, :3].reshape(1, H, W, 3).transpose(0, 3, 1, 2)
    return img
